# baseline (speedup 1.0000x reference)
import functools

import jax
import jax.numpy as jnp
from jax.experimental import pallas as pl
from jax.experimental.pallas import tpu as pltpu
from jax.experimental.pallas import tpu_sc as plsc

BATCH = 4096
SEQ = 200
EMBED = 64
LANES = 16
BTILE = 128
NBT = BATCH // BTILE
NWIN = SEQ * NBT
NWORK = 32
STEPS = NWIN // NWORK


def _prep_body(seg_table_ref, pos_table_ref, comb_ref):
    pos = pos_table_ref[...]
    seg = seg_table_ref[...]
    comb_ref[...] = pos + seg[:, None, :]


_prep = pl.pallas_call(
    _prep_body,
    out_shape=jax.ShapeDtypeStruct((3, SEQ, EMBED), jnp.float32),
)

VOCAB = 1000000
TBLK = 1024


def _pair_body(in_ref, out_ref):
    x = in_ref[...]
    xt = jnp.transpose(x)
    half = TBLK // 2
    out_ref[...] = jnp.concatenate([xt[:half, :], xt[half:, :]], axis=1)


_pair = pl.pallas_call(
    _pair_body,
    grid=(pl.cdiv(VOCAB, TBLK),),
    in_specs=[pl.BlockSpec((EMBED, TBLK), lambda i: (0, i))],
    out_specs=pl.BlockSpec((TBLK // 2, 128), lambda i: (i, 0)),
    out_shape=jax.ShapeDtypeStruct((VOCAB // 2, 128), jnp.float32),
)


def _embed_kernel(tok_hbm, comb_hbm, seqt_hbm, segt_hbm, out_hbm,
                  comb_tbl,
                  sq0, sq1, sg0, sg1, cr0, cr1, si0, si1,
                  t0, t1, ob0, ob1,
                  sem_i0, sem_i1, sem_g0, sem_g1, sem_o0, sem_o1):
    pltpu.sync_copy(comb_hbm, comb_tbl)

    wid = jax.lax.axis_index("subcore") * 2 + jax.lax.axis_index("core")
    base = wid * STEPS

    iota = jax.lax.iota(jnp.int32, LANES)

    slots = (
        (sq0, sg0, cr0, si0, t0, ob0, sem_i0, sem_g0, sem_o0),
        (sq1, sg1, cr1, si1, t1, ob1, sem_i1, sem_g1, sem_o1),
    )

    def win_coords(k):
        w = base + k
        return w // NBT, w % NBT

    def issue_idx(k, slot):
        sq, sg, _, _, _, _, sem_i, _, _ = slot
        s, bt = win_coords(k)
        b0 = bt * BTILE
        pltpu.async_copy(seqt_hbm.at[s, pl.ds(b0, BTILE)], sq, sem_i)
        pltpu.async_copy(segt_hbm.at[s, pl.ds(b0, BTILE)], sg, sem_i)

    def wait_idx(slot):
        sq, sg, _, _, _, _, sem_i, _, _ = slot
        pltpu.make_async_copy(seqt_hbm.at[0, pl.ds(0, BTILE)], sq, sem_i).wait()
        pltpu.make_async_copy(segt_hbm.at[0, pl.ds(0, BTILE)], sg, sem_i).wait()

    def issue_gather(slot):
        sq, _, _, si, t, _, _, sem_g, _ = slot
        for g in range(BTILE // LANES):
            gsl = pl.ds(g * LANES, LANES)
            v = sq[gsl]
            si[gsl] = jax.lax.bitwise_or(
                jax.lax.shift_left(
                    jax.lax.shift_right_logical(v, 10), 9),
                jax.lax.bitwise_and(v, 511),
            )
        pltpu.async_copy(tok_hbm.at[si], t, sem_g)

    def wait_gather(slot):
        _, _, _, si, t, _, _, sem_g, _ = slot
        pltpu.make_async_copy(tok_hbm.at[si], t, sem_g).wait()

    def out_slice(k):
        s, bt = win_coords(k)
        return out_hbm.at[s, :, bt]

    def wait_out(k, slot):
        pltpu.make_async_copy(slot[5], out_slice(k), slot[8]).wait()

    for kp in range(2):
        sq, sg, _, _, _, _, _, _, _ = slots[kp]
        s, bt = win_coords(kp)
        b0 = bt * BTILE
        pltpu.sync_copy(seqt_hbm.at[s, pl.ds(b0, BTILE)], sq)
        pltpu.sync_copy(segt_hbm.at[s, pl.ds(b0, BTILE)], sg)
        issue_gather(slots[kp])

    def stage(k, slot_i):
        slot = slots[slot_i]
        sq, sg, cr, si, t, ob, sem_i, sem_g, sem_o = slot
        s, bt = win_coords(k)
        wait_gather(slot)

        @pl.when(k >= 2)
        def _():
            pltpu.make_async_copy(ob, out_slice(k), sem_o).wait()

        for g in range(BTILE // LANES):
            gsl = pl.ds(g * LANES, LANES)
            cr[gsl] = sg[gsl] * SEQ + s

        @pl.loop(0, LANES)
        def _(j):
            jm = jax.lax.bitwise_and(iota + j, LANES - 1)
            for r0 in range(BTILE // LANES):
                r_v = r0 * LANES + iota
                gsl = pl.ds(r0 * LANES, LANES)
                crow = cr[gsl]
                h64 = jax.lax.shift_left(
                    jax.lax.bitwise_and(
                        jax.lax.shift_right_logical(sq[gsl], 9), 1), 6)
                for c0 in range(EMBED // LANES):
                    c_v = c0 * LANES + jm
                    tv = plsc.load_gather(t, [r_v, h64 + c_v])
                    cv = plsc.load_gather(comb_tbl, [crow, c_v])
                    plsc.store_scatter(
                        ob,
                        [jax.lax.shift_right_logical(c_v, 3),
                         jax.lax.bitwise_and(c_v, 7),
                         r_v],
                        tv + cv,
                    )

        pltpu.async_copy(ob, out_slice(k), sem_o)

        @pl.when(k + 2 < STEPS)
        def _():
            issue_idx(k + 2, slot)
            wait_idx(slot)
            issue_gather(slot)

    @pl.loop(0, STEPS, step=2)
    def _(k):
        stage(k, 0)
        stage(k + 1, 1)

    wait_out(STEPS - 2, slots[0])
    wait_out(STEPS - 1, slots[1])


def _make_embed():
    mesh = plsc.VectorSubcoreMesh(
        core_axis_name="core", subcore_axis_name="subcore"
    )
    return pl.kernel(
        _embed_kernel,
        out_type=jax.ShapeDtypeStruct(
            (SEQ, EMBED // 8, NBT, 8, BTILE), jnp.float32),
        mesh=mesh,
        compiler_params=pltpu.CompilerParams(
            use_tc_tiling_on_sc=False, needs_layout_passes=False),
        scratch_types=[
            pltpu.VMEM((3 * SEQ, EMBED), jnp.float32),
            pltpu.VMEM((BTILE,), jnp.int32),
            pltpu.VMEM((BTILE,), jnp.int32),
            pltpu.VMEM((BTILE,), jnp.int32),
            pltpu.VMEM((BTILE,), jnp.int32),
            pltpu.VMEM((BTILE,), jnp.int32),
            pltpu.VMEM((BTILE,), jnp.int32),
            pltpu.VMEM((BTILE,), jnp.int32),
            pltpu.VMEM((BTILE,), jnp.int32),
            pltpu.VMEM((BTILE, 128), jnp.float32),
            pltpu.VMEM((BTILE, 128), jnp.float32),
            pltpu.VMEM((EMBED // 8, 8, BTILE), jnp.float32),
            pltpu.VMEM((EMBED // 8, 8, BTILE), jnp.float32),
        ] + [pltpu.SemaphoreType.DMA] * 6,
    )


_embed = _make_embed()


@jax.jit
def kernel(sequence, segment_label, token_table, segment_table, pos_table):
    comb = _prep(segment_table, pos_table)
    out5 = _embed(
        _pair(token_table.T),
        comb.reshape(3 * SEQ, EMBED),
        sequence.astype(jnp.int32).T,
        segment_label.astype(jnp.int32).T,
    )
    return out5.transpose(2, 4, 0, 1, 3).reshape(BATCH, SEQ, EMBED)


# speedup vs baseline: 1.1846x; 1.1846x over previous
import functools

import jax
import jax.numpy as jnp
from jax.experimental import pallas as pl
from jax.experimental.pallas import tpu as pltpu
from jax.experimental.pallas import tpu_sc as plsc

BATCH = 4096
SEQ = 200
EMBED = 64
NFLAT = BATCH * SEQ
WIN = 128
LANES = 16
NWORK = 32
STEPS = NFLAT // WIN // NWORK


def _prep_body(seg_label_ref, seg_table_ref, pos_table_ref, comb_ref, cidx_ref):
    pos = pos_table_ref[...]
    seg = seg_table_ref[...]
    comb_ref[...] = pos + seg[:, None, :]
    s_iota = jax.lax.broadcasted_iota(jnp.int32, cidx_ref.shape, 1)
    cidx_ref[...] = seg_label_ref[...] * SEQ + s_iota


_prep = pl.pallas_call(
    _prep_body,
    out_shape=[
        jax.ShapeDtypeStruct((3, SEQ, EMBED), jnp.float32),
        jax.ShapeDtypeStruct((BATCH, SEQ), jnp.int32),
    ],
)


def _embed_kernel(tok_hbm, comb_hbm, seq_hbm, cidx_hbm, out_hbm,
                  comb_tbl,
                  idx_a0, idx_a1, idx_b0, idx_b1,
                  tok0, tok1, cmb0, cmb1, ob0, ob1,
                  sem_i0, sem_i1, sem_gt0, sem_gt1,
                  sem_gc0, sem_gc1, sem_o0, sem_o1):
    @pl.when(jax.lax.axis_index("subcore") == 0)
    def _():
        pltpu.sync_copy(comb_hbm, comb_tbl)

    plsc.subcore_barrier()

    wid = jax.lax.axis_index("subcore") * 2 + jax.lax.axis_index("core")
    base = wid * STEPS

    slots = (
        (idx_a0, idx_b0, tok0, cmb0, ob0, sem_i0, sem_gt0, sem_gc0, sem_o0),
        (idx_a1, idx_b1, tok1, cmb1, ob1, sem_i1, sem_gt1, sem_gc1, sem_o1),
    )

    def issue_gathers(slot):
        idx_a, idx_b, tok, cmb, _, _, sem_gt, sem_gc, _ = slot
        pltpu.async_copy(tok_hbm.at[idx_a], tok, sem_gt)
        pltpu.async_copy(comb_tbl.at[idx_b], cmb, sem_gc)

    def wait_gathers(slot):
        idx_a, idx_b, tok, cmb, _, _, sem_gt, sem_gc, _ = slot
        pltpu.make_async_copy(tok_hbm.at[idx_a], tok, sem_gt).wait()
        pltpu.make_async_copy(comb_tbl.at[idx_b], cmb, sem_gc).wait()

    def issue_idx(k, slot):
        idx_a, idx_b, _, _, _, sem_i, _, _, _ = slot
        off = (base + k) * WIN
        pltpu.async_copy(seq_hbm.at[pl.ds(off, WIN)], idx_a, sem_i)
        pltpu.async_copy(cidx_hbm.at[pl.ds(off, WIN)], idx_b, sem_i)

    def wait_idx(slot):
        idx_a, idx_b, _, _, _, sem_i, _, _, _ = slot
        pltpu.make_async_copy(seq_hbm.at[pl.ds(0, WIN)], idx_a, sem_i).wait()
        pltpu.make_async_copy(cidx_hbm.at[pl.ds(0, WIN)], idx_b, sem_i).wait()

    def wait_out(k, slot):
        _, _, _, _, ob, _, _, _, sem_o = slot
        row0 = (base + k) * WIN
        pltpu.make_async_copy(
            ob, out_hbm.at[pl.ds(row0, WIN)], sem_o
        ).wait()

    for s in range(2):
        idx_a, idx_b, _, _, _, _, _, _, _ = slots[s]
        off = (base + s) * WIN
        pltpu.sync_copy(seq_hbm.at[pl.ds(off, WIN)], idx_a)
        pltpu.sync_copy(cidx_hbm.at[pl.ds(off, WIN)], idx_b)
        issue_gathers(slots[s])

    def stage(k, s):
        slot = slots[s]
        _, _, tok, cmb, ob, _, _, _, sem_o = slot
        wait_gathers(slot)

        @pl.when(k + 2 < STEPS)
        def _():
            issue_idx(k + 2, slot)

        @pl.when(k >= 2)
        def _():
            wait_out(k - 2, slot)

        @pl.loop(0, WIN, step=4)
        def _(r):
            for rr in range(4):
                for j in range(EMBED // LANES):
                    slc = (pl.ds(r + rr, 1), pl.ds(j * LANES, LANES))
                    ob.at[slc][...] = tok.at[slc][...] + cmb.at[slc][...]

        row0 = (base + k) * WIN
        pltpu.async_copy(ob, out_hbm.at[pl.ds(row0, WIN)], sem_o)

        @pl.when(k + 2 < STEPS)
        def _():
            wait_idx(slot)
            issue_gathers(slot)

    @pl.loop(0, STEPS, step=2)
    def _(k):
        stage(k, 0)
        stage(k + 1, 1)

    wait_out(STEPS - 2, slots[0])
    wait_out(STEPS - 1, slots[1])


def _make_embed():
    mesh = plsc.VectorSubcoreMesh(
        core_axis_name="core", subcore_axis_name="subcore"
    )
    return pl.kernel(
        _embed_kernel,
        out_type=jax.ShapeDtypeStruct((NFLAT, EMBED), jnp.float32),
        mesh=mesh,
        compiler_params=pltpu.CompilerParams(use_tc_tiling_on_sc=False),
        scratch_types=[
            pltpu.VMEM_SHARED((3 * SEQ, EMBED), jnp.float32),
            pltpu.VMEM((WIN,), jnp.int32),
            pltpu.VMEM((WIN,), jnp.int32),
            pltpu.VMEM((WIN,), jnp.int32),
            pltpu.VMEM((WIN,), jnp.int32),
            pltpu.VMEM((WIN, EMBED), jnp.float32),
            pltpu.VMEM((WIN, EMBED), jnp.float32),
            pltpu.VMEM((WIN, EMBED), jnp.float32),
            pltpu.VMEM((WIN, EMBED), jnp.float32),
            pltpu.VMEM((WIN, EMBED), jnp.float32),
            pltpu.VMEM((WIN, EMBED), jnp.float32),
        ] + [pltpu.SemaphoreType.DMA] * 8,
    )


_embed = _make_embed()


@jax.jit
def kernel(sequence, segment_label, token_table, segment_table, pos_table):
    comb, cidx = _prep(
        segment_label.astype(jnp.int32), segment_table, pos_table
    )
    out = _embed(
        token_table,
        comb.reshape(3 * SEQ, EMBED),
        sequence.astype(jnp.int32).reshape(NFLAT),
        cidx.reshape(NFLAT),
    )
    return out.reshape(BATCH, SEQ, EMBED)
